# baseline (speedup 1.0000x reference)
import functools
import math

import jax
import jax.numpy as jnp
from jax import lax
from jax.experimental import pallas as pl
from jax.experimental.pallas import tpu as pltpu
from jax.experimental.pallas import tpu_sc as plsc

F32 = jnp.float32
_EPS = 1e-5
_NC, _NS = 2, 16
_NW = _NC * _NS
_CE = 80
_NCH = 2
_NB = 1000
_EB = 4096


def _ln(h, g, beta):
    mu = jnp.mean(h, axis=-1, keepdims=True)
    var = jnp.mean((h - mu) ** 2, axis=-1, keepdims=True)
    return (h - mu) * lax.rsqrt(var + _EPS) * g + beta


def _full(shape):
    return pl.BlockSpec(shape, lambda i: tuple(0 for _ in shape))


def _enc_apply(xp, w1t, b1, w2t, b2, g, be, nb):
    r, din = xp.shape

    def body(x_ref, w1_ref, b1_ref, w2_ref, b2_ref, g_ref, be_ref, o_ref):
        h = jnp.maximum(
            jnp.dot(x_ref[...], w1_ref[...], preferred_element_type=F32)
            + b1_ref[...], 0.0)
        h = jnp.dot(h, w2_ref[...], preferred_element_type=F32) + b2_ref[...]
        o_ref[...] = _ln(h, g_ref[...], be_ref[...])

    return pl.pallas_call(
        body, grid=(r // nb,),
        in_specs=[pl.BlockSpec((nb, din), lambda i: (i, 0)),
                  _full(w1t.shape), _full(b1.shape), _full(w2t.shape),
                  _full(b2.shape), _full(g.shape), _full(be.shape)],
        out_specs=pl.BlockSpec((nb, 128), lambda i: (i, 0)),
        out_shape=jax.ShapeDtypeStruct((r, 128), F32),
    )(xp, w1t, b1, w2t, b2, g, be)


def _vt_x(v, xe):
    n = v.shape[0]

    def body(v_ref, x_ref, o_ref):
        @pl.when(pl.program_id(0) == 0)
        def _():
            o_ref[...] = jnp.zeros_like(o_ref)
        o_ref[...] += lax.dot_general(
            v_ref[...], x_ref[...], (((0,), (0,)), ((), ())),
            preferred_element_type=F32)

    return pl.pallas_call(
        body, grid=(n // _NB,),
        in_specs=[pl.BlockSpec((_NB, 128), lambda i: (i, 0)),
                  pl.BlockSpec((_NB, 128), lambda i: (i, 0))],
        out_specs=pl.BlockSpec((128, 128), lambda i: (0, 0)),
        out_shape=jax.ShapeDtypeStruct((128, 128), F32),
        compiler_params=pltpu.CompilerParams(
            dimension_semantics=("arbitrary",)),
    )(v, xe)


def _v_s(v, s):
    n = v.shape[0]

    def body(v_ref, s_ref, o_ref):
        o_ref[...] = jnp.dot(v_ref[...], s_ref[...],
                             preferred_element_type=F32)

    return pl.pallas_call(
        body, grid=(n // _NB,),
        in_specs=[pl.BlockSpec((_NB, 128), lambda i: (i, 0)), _full((128, 128))],
        out_specs=pl.BlockSpec((_NB, 128), lambda i: (i, 0)),
        out_shape=jax.ShapeDtypeStruct((n, 128), F32),
    )(v, s)


def _ab_tables(xs, p, wxa, wpa, ba, wxb, wpb):
    n = xs.shape[0]

    def body(x_ref, p_ref, wxa_r, wpa_r, ba_r, wxb_r, wpb_r, a_ref, b_ref):
        x = x_ref[...]
        pp = p_ref[...]
        a_ref[...] = (
            jnp.dot(x, wxa_r[...], preferred_element_type=F32)
            + jnp.dot(pp, wpa_r[...], preferred_element_type=F32)
            + ba_r[...])
        b_ref[...] = (
            jnp.dot(x, wxb_r[...], preferred_element_type=F32)
            + jnp.dot(pp, wpb_r[...], preferred_element_type=F32))

    blk = pl.BlockSpec((_NB, 128), lambda i: (i, 0))
    return pl.pallas_call(
        body, grid=(n // _NB,),
        in_specs=[blk, blk, _full((128, 128)), _full((128, 128)),
                  _full((1, 128)), _full((128, 128)), _full((128, 128))],
        out_specs=[blk, blk],
        out_shape=[jax.ShapeDtypeStruct((n, 128), F32)] * 2,
    )(xs, p, wxa, wpa, ba, wxb, wpb)


def _spc_layer1(sx1, sea_raw, mdst, msrc, msrct, ew, pw):
    def body(sx_ref, sraw_ref, mdst_ref, msrc_ref, msrct_ref,
             ew1t, eb1, ew2t, eb2, eg, ebe,
             wit, wjt, wet, pb1, pw2t, pb2, pg, pbe,
             wnxt, wnat, bn, wn2t, bn2, gn, ben,
             o_ref):
        sx = sx_ref[...]
        sea = jnp.maximum(
            jnp.dot(sraw_ref[...], ew1t[...], preferred_element_type=F32)
            + eb1[...], 0.0)
        sea = jnp.dot(sea, ew2t[...], preferred_element_type=F32) + eb2[...]
        sea = _ln(sea, eg[...], ebe[...])
        xiw = jnp.dot(sx, wit[...], preferred_element_type=F32)
        xjw = jnp.dot(sx, wjt[...], preferred_element_type=F32)
        pre = (jnp.dot(mdst_ref[...], xiw, preferred_element_type=F32)
               + jnp.dot(msrc_ref[...], xjw, preferred_element_type=F32)
               + jnp.dot(sea, wet[...], preferred_element_type=F32)
               + pb1[...])
        h = jnp.maximum(pre, 0.0)
        h = jnp.dot(h, pw2t[...], preferred_element_type=F32) + pb2[...]
        upd = _ln(h, pg[...], pbe[...]) + sea
        agg = jnp.dot(msrct_ref[...], upd, preferred_element_type=F32)
        pre_n = (jnp.dot(sx, wnxt[...], preferred_element_type=F32)
                 + jnp.dot(agg, wnat[...], preferred_element_type=F32)
                 + bn[...])
        hn = jnp.maximum(pre_n, 0.0)
        hn = jnp.dot(hn, wn2t[...], preferred_element_type=F32) + bn2[...]
        o_ref[...] = sx[:, :128] + _ln(hn, gn[...], ben[...])

    args = (sx1, sea_raw, mdst, msrc, msrct) + ew + pw
    return pl.pallas_call(
        body, out_shape=jax.ShapeDtypeStruct((128, 128), F32),
    )(*args)


def _edge_mlp(ea_in, g, wenc, wet, w2t, b2, gg, be):
    e = ea_in.shape[0]

    def body(*refs):
        if wenc is not None:
            (ea_ref, g_ref, e1t, eb1, e2t, eb2, eg, ebe,
             wet_r, w2t_r, b2_r, g_r, be_r, o_ref) = refs
            ea = jnp.maximum(
                jnp.dot(ea_ref[...], e1t[...], preferred_element_type=F32)
                + eb1[...], 0.0)
            ea = jnp.dot(ea, e2t[...], preferred_element_type=F32) + eb2[...]
            ea = _ln(ea, eg[...], ebe[...])
        else:
            (ea_ref, g_ref, wet_r, w2t_r, b2_r, g_r, be_r, o_ref) = refs
            ea = ea_ref[...]
        pre = (g_ref[...]
               + jnp.dot(ea, wet_r[...], preferred_element_type=F32))
        h = jnp.maximum(pre, 0.0)
        h = jnp.dot(h, w2t_r[...], preferred_element_type=F32) + b2_r[...]
        o_ref[...] = _ln(h, g_r[...], be_r[...]) + ea

    w = ea_in.shape[1]
    in_specs = [pl.BlockSpec((_EB, w), lambda i: (i, 0)),
                pl.BlockSpec((_EB, 128), lambda i: (i, 0))]
    args = [ea_in, g]
    if wenc is not None:
        for a in wenc:
            in_specs.append(_full(a.shape))
            args.append(a)
    for a in (wet, w2t, b2, gg, be):
        in_specs.append(_full(a.shape))
        args.append(a)
    return pl.pallas_call(
        body, grid=(e // _EB,),
        in_specs=in_specs,
        out_specs=pl.BlockSpec((_EB, 128), lambda i: (i, 0)),
        out_shape=jax.ShapeDtypeStruct((e, 128), F32),
    )(*args)


def _node_mlp(xs, p, parts, f, nw, extra, mode):
    n = xs.shape[0]
    np_parts = parts.shape[0]
    wnxt, wnpt, wnat, wnft, bn, wn2t, bn2, gn, ben = nw

    def body(*refs):
        xs_ref, p_ref = refs[0], refs[1]
        prefs = refs[2:2 + np_parts]
        f_ref = refs[2 + np_parts]
        rest = refs[3 + np_parts:]
        if mode == "dec":
            r_ref = rest[0]
            rest = rest[1:]
        (wnxt_r, wnpt_r, wnat_r, wnft_r, bn_r,
         wn2t_r, bn2_r, gn_r, ben_r) = rest[:9]
        wext = rest[9:14]
        outs = rest[14:]
        xsv = xs_ref[...]
        pv = p_ref[...]
        agg = prefs[0][0]
        for pr in prefs[1:]:
            agg = agg + pr[0]
        pre = (jnp.dot(xsv, wnxt_r[...], preferred_element_type=F32)
               + jnp.dot(pv, wnpt_r[...], preferred_element_type=F32)
               + jnp.dot(agg, wnat_r[...], preferred_element_type=F32)
               + jnp.dot(f_ref[...], wnft_r[...], preferred_element_type=F32)
               + bn_r[...])
        h = jnp.maximum(pre, 0.0)
        h = jnp.dot(h, wn2t_r[...], preferred_element_type=F32) + bn2_r[...]
        xi = xsv + _ln(h, gn_r[...], ben_r[...])
        if mode == "ab":
            wxa, wpa, ba, wxb, wpb = wext
            xi_ref, a_ref, b_ref = outs
            xi_ref[...] = xi
            a_ref[...] = (
                jnp.dot(xi, wxa[...], preferred_element_type=F32)
                + jnp.dot(pv, wpa[...], preferred_element_type=F32)
                + ba[...])
            b_ref[...] = (
                jnp.dot(xi, wxb[...], preferred_element_type=F32)
                + jnp.dot(pv, wpb[...], preferred_element_type=F32))
        else:
            wd1xt, wd1rt, bd1, wd2t, bd2 = wext
            o_ref, = outs
            hd = jnp.maximum(
                jnp.dot(xi, wd1xt[...], preferred_element_type=F32)
                + jnp.dot(r_ref[...], wd1rt[...], preferred_element_type=F32)
                + bd1[...], 0.0)
            o_ref[...] = (jnp.dot(hd, wd2t[...], preferred_element_type=F32)
                          + bd2[...])

    blk = pl.BlockSpec((_NB, 128), lambda i: (i, 0))
    args = [xs, p] + [parts] * np_parts + [f]
    in_specs = [blk, blk]
    for j in range(np_parts):
        in_specs.append(
            pl.BlockSpec((1, _NB, 128),
                         functools.partial(lambda i, jj: (jj, i, 0), jj=j)))
    in_specs.append(blk)
    if mode == "dec":
        args.append(extra[0])
        in_specs.append(blk)
        wlist = nw + tuple(extra[1:])
    else:
        wlist = nw + tuple(extra)
    for a in wlist:
        args.append(a)
        in_specs.append(_full(a.shape))
    if mode == "ab":
        out_specs = [blk, blk, blk]
        out_shape = [jax.ShapeDtypeStruct((n, 128), F32)] * 3
    else:
        out_specs = blk
        out_shape = jax.ShapeDtypeStruct((n, 128), F32)
    return pl.pallas_call(
        body, grid=(n // _NB,),
        in_specs=in_specs, out_specs=out_specs, out_shape=out_shape,
    )(*args)


def _sc_gather(a_tab, b_tab, dst, src):
    e = dst.shape[0]
    per_w = e // _NW
    nck = per_w // _CE
    mesh = plsc.VectorSubcoreMesh(core_axis_name="c", subcore_axis_name="s")

    @functools.partial(
        pl.kernel, mesh=mesh,
        out_type=jax.ShapeDtypeStruct((e, 128), F32),
        scratch_types=[
            pltpu.VMEM((per_w,), jnp.int32),
            pltpu.VMEM((per_w,), jnp.int32),
            pltpu.VMEM((2, _CE, 128), F32),
            pltpu.VMEM((2, _CE, 128), F32),
            pltpu.SemaphoreType.DMA,
            pltpu.SemaphoreType.DMA,
            pltpu.SemaphoreType.DMA,
            pltpu.SemaphoreType.DMA,
        ],
    )
    def k(a_h, b_h, dst_h, src_h, o_h,
          idxd, idxs, bufa, bufb, sg0, sg1, sw0, sw1):
        wid = lax.axis_index("s") * _NC + lax.axis_index("c")
        base = wid * per_w
        pltpu.sync_copy(dst_h.at[pl.ds(base, per_w)], idxd)
        pltpu.sync_copy(src_h.at[pl.ds(base, per_w)], idxs)
        sgs = (sg0, sg1)
        sws = (sw0, sw1)

        def issue(ci, p):
            off = ci * _CE
            pltpu.async_copy(a_h.at[idxd.at[pl.ds(off, _CE)]], bufa.at[p],
                             sgs[p])
            pltpu.async_copy(b_h.at[idxs.at[pl.ds(off, _CE)]], bufb.at[p],
                             sgs[p])

        def wait_g(p):
            pltpu.make_async_copy(a_h.at[idxd.at[pl.ds(0, _CE)]], bufa.at[p],
                                  sgs[p]).wait()
            pltpu.make_async_copy(b_h.at[idxs.at[pl.ds(0, _CE)]], bufb.at[p],
                                  sgs[p]).wait()

        def add(p):
            def row(r, carry):
                for j in range(8):
                    sl = pl.ds(j * 16, 16)
                    bufa[p, r, sl] = bufa[p, r, sl] + bufb[p, r, sl]
                return carry

            lax.fori_loop(0, _CE, row, 0)

        def write(ci, p):
            off = base + ci * _CE
            pltpu.async_copy(bufa.at[p], o_h.at[pl.ds(off, _CE)], sws[p])

        def wait_w(p):
            pltpu.make_async_copy(bufa.at[p], o_h.at[pl.ds(base, _CE)],
                                  sws[p]).wait()

        issue(0, 0)
        issue(1, 1)

        def step(ci, carry):
            wait_g(0)
            add(0)
            write(ci, 0)
            wait_g(1)
            add(1)
            write(ci + 1, 1)
            wait_w(0)
            issue(ci + 2, 0)
            wait_w(1)

            @pl.when(ci + 3 < nck)
            def _():
                issue(ci + 3, 1)

            return carry

        if nck % 2:
            lax.fori_loop(0, (nck - 1) // 2, lambda i, c: step(2 * i, c), 0)
            wait_g(0)
            add(0)
            write(nck - 1, 0)
            wait_w(0)
        else:
            lax.fori_loop(0, (nck - 2) // 2, lambda i, c: step(2 * i, c), 0)
            wait_g(0)
            add(0)
            write(nck - 2, 0)
            wait_g(1)
            add(1)
            write(nck - 1, 1)
            wait_w(0)
            wait_w(1)

    return k(a_tab, b_tab, dst, src)


def _sc_scatter(upd, src3d, n_pad):
    rows_w = src3d.shape[1]
    rps = n_pad // _NS
    zr = 32
    mesh = plsc.VectorSubcoreMesh(core_axis_name="c", subcore_axis_name="s")

    @functools.partial(
        pl.kernel, mesh=mesh,
        out_type=jax.ShapeDtypeStruct((_NC, n_pad, 128), F32),
        scratch_types=[
            pltpu.VMEM((rows_w, _CE), jnp.int32),
            pltpu.VMEM((2, _CE, 128), F32),
            pltpu.VMEM((zr, 128), F32),
            pltpu.VMEM_SHARED((n_pad, 128), F32),
            pltpu.SemaphoreType.DMA,
            pltpu.SemaphoreType.DMA,
            pltpu.SemaphoreType.DMA,
            pltpu.SemaphoreType.DMA,
        ],
    )
    def k(upd_h, src_h, out_h, idx2d, rows, stage, acc, si0, si1, ss0, ss1):
        c = lax.axis_index("c")
        s = lax.axis_index("s")
        wid = s * _NC + c
        sis = (si0, si1)
        sss = (ss0, ss1)

        def zrow(r, carry):
            for j in range(8):
                stage[r, pl.ds(j * 16, 16)] = jnp.zeros((16,), F32)
            return carry

        lax.fori_loop(0, zr, zrow, 0)

        def zcopy(t, carry):
            pltpu.sync_copy(stage, acc.at[pl.ds(s * rps + t * zr, zr)])
            return carry

        lax.fori_loop(0, rps // zr, zcopy, 0)
        pltpu.sync_copy(src_h.at[wid], idx2d)
        plsc.subcore_barrier()

        base = wid * rows_w * _CE

        def issue_in(ci, p):
            pltpu.async_copy(upd_h.at[pl.ds(base + ci * _CE, _CE)],
                             rows.at[p], sis[p])

        def wait_in(p):
            pltpu.make_async_copy(upd_h.at[pl.ds(base, _CE)], rows.at[p],
                                  sis[p]).wait()

        def issue_sc(ci, p):
            pltpu.async_copy(rows.at[p], acc.at[idx2d.at[ci]], sss[p],
                             add=True)

        def wait_sc(ci, p):
            pltpu.make_async_copy(rows.at[p], acc.at[idx2d.at[ci]],
                                  sss[p]).wait()

        issue_in(0, 0)
        issue_in(1, 1)

        def step(ci, carry):
            wait_in(0)
            issue_sc(ci, 0)
            wait_in(1)
            issue_sc(ci + 1, 1)
            wait_sc(ci, 0)
            issue_in(ci + 2, 0)
            wait_sc(ci + 1, 1)

            @pl.when(ci + 3 < rows_w)
            def _():
                issue_in(ci + 3, 1)

            return carry

        if rows_w % 2:
            lax.fori_loop(0, (rows_w - 1) // 2,
                          lambda i, cr: step(2 * i, cr), 0)
            wait_in(0)
            issue_sc(rows_w - 1, 0)
            wait_sc(rows_w - 1, 0)
        else:
            lax.fori_loop(0, (rows_w - 2) // 2,
                          lambda i, cr: step(2 * i, cr), 0)
            wait_in(0)
            issue_sc(rows_w - 2, 0)
            wait_in(1)
            issue_sc(rows_w - 1, 1)
            wait_sc(rows_w - 2, 0)
            wait_sc(rows_w - 1, 1)
        plsc.subcore_barrier()

        def wback(t, carry):
            r0 = s * rps + t * zr
            pltpu.sync_copy(acc.at[pl.ds(r0, zr)], stage)
            pltpu.sync_copy(stage, out_h.at[c, pl.ds(r0, zr)])
            return carry

        lax.fori_loop(0, rps // zr, wback, 0)

    return k(upd, src3d)


def _enc_w(p, pad_to=None):
    w1t = p["l1"]["W"].T
    if pad_to is not None:
        w1t = jnp.pad(w1t, ((0, pad_to - w1t.shape[0]), (0, 0)))
    return (w1t, p["l1"]["b"][None], p["l2"]["W"].T, p["l2"]["b"][None],
            p["g"][None], p["beta"][None])


def kernel(x, edge_attr, del_ext_force, spc_x, spc_edge_attr, V, params,
           edge_index, spc_edge_index):
    n = x.shape[0]
    n_pad = ((n + 16 * 128 - 1) // (16 * 128)) * (16 * 128)
    k_dim = spc_x.shape[0]
    e = edge_index.shape[1]
    align = (_NW * _CE) * _EB // math.gcd(_NW * _CE, _EB)
    ec = ((e // _NCH + align - 1) // align) * align
    epad = _NCH * ec - e
    idx = edge_index.astype(jnp.int32)
    srcp = jnp.concatenate(
        [idx[0], n + (jnp.arange(epad, dtype=jnp.int32) % (n_pad - n))])
    dstp = jnp.concatenate([idx[1], jnp.zeros((epad,), jnp.int32)])
    ea_p = jnp.pad(edge_attr, ((0, epad), (0, 0)))
    srcc = [srcp[c * ec:(c + 1) * ec] for c in range(_NCH)]
    dstc = [dstp[c * ec:(c + 1) * ec] for c in range(_NCH)]
    src3dc = [s.reshape(_NW, -1, _CE) for s in srcc]

    f_pad = jnp.pad(del_ext_force, ((0, 0), (0, 128 - del_ext_force.shape[1])))
    spc_x_pad = jnp.pad(spc_x, ((0, 0), (0, 128 - spc_x.shape[1])))
    sea_raw = jnp.pad(spc_edge_attr, ((0, 0), (0, 128 - spc_edge_attr.shape[1])))

    kk = jnp.arange(k_dim, dtype=jnp.int32)
    mdst = (spc_edge_index[1][:, None] == kk[None, :]).astype(F32)
    msrc = (spc_edge_index[0][:, None] == kk[None, :]).astype(F32)
    msrct = (kk[:, None] == spc_edge_index[0][None, :]).astype(F32)

    xe = _enc_apply(x, *_enc_w(params["node_encoder"]), _NB)
    se = _enc_apply(spc_x_pad, *_enc_w(params["spc_node_encoder"], 128), 128)

    q = _vt_x(V, xe)
    p_tab = _v_s(V, se)
    sx1 = jnp.concatenate([se, q], axis=1)

    sp = params["spc_proc"][0]
    sw1 = sp["edge_mlp"]["l1"]["W"]
    snw1 = sp["node_mlp"]["l1"]["W"]
    sxi1 = _spc_layer1(
        sx1, sea_raw, mdst, msrc, msrct,
        _enc_w(params["spc_edge_encoder"], 128),
        (sw1[:, :256].T, sw1[:, 256:512].T, sw1[:, 512:].T,
         sp["edge_mlp"]["l1"]["b"][None],
         sp["edge_mlp"]["l2"]["W"].T, sp["edge_mlp"]["l2"]["b"][None],
         sp["edge_mlp"]["g"][None], sp["edge_mlp"]["beta"][None],
         snw1[:, :256].T, snw1[:, 256:].T, sp["node_mlp"]["l1"]["b"][None],
         sp["node_mlp"]["l2"]["W"].T, sp["node_mlp"]["l2"]["b"][None],
         sp["node_mlp"]["g"][None], sp["node_mlp"]["beta"][None]))
    r_tab = _v_s(V, sxi1)

    p0 = params["proc"][0]
    w1 = p0["edge_mlp"]["l1"]["W"]
    a1, b1 = _ab_tables(xe, p_tab, w1[:, :128].T, w1[:, 128:256].T,
                        p0["edge_mlp"]["l1"]["b"][None],
                        w1[:, 256:384].T, w1[:, 384:512].T)
    upd1c = []
    part1s = []
    for c in range(_NCH):
        g1 = _sc_gather(a1, b1, dstc[c], srcc[c])
        u = _edge_mlp(ea_p[c * ec:(c + 1) * ec], g1,
                      _enc_w(params["edge_encoder"]),
                      w1[:, 512:].T, p0["edge_mlp"]["l2"]["W"].T,
                      p0["edge_mlp"]["l2"]["b"][None],
                      p0["edge_mlp"]["g"][None],
                      p0["edge_mlp"]["beta"][None])
        upd1c.append(u)
        part1s.append(_sc_scatter(u, src3dc[c], n_pad))
    part1 = jnp.concatenate(part1s, axis=0)

    p1 = params["proc"][1]
    w1b = p1["edge_mlp"]["l1"]["W"]
    nw0 = p0["node_mlp"]["l1"]["W"]
    nodew0 = (nw0[:, :128].T, nw0[:, 128:256].T, nw0[:, 256:384].T,
              jnp.pad(nw0[:, 384:].T, ((0, 125), (0, 0))),
              p0["node_mlp"]["l1"]["b"][None],
              p0["node_mlp"]["l2"]["W"].T, p0["node_mlp"]["l2"]["b"][None],
              p0["node_mlp"]["g"][None], p0["node_mlp"]["beta"][None])
    xi1, a2, b2 = _node_mlp(
        xe, p_tab, part1, f_pad, nodew0,
        (w1b[:, :128].T, w1b[:, 128:256].T, p1["edge_mlp"]["l1"]["b"][None],
         w1b[:, 256:384].T, w1b[:, 384:512].T), "ab")

    part2s = []
    for c in range(_NCH):
        g2 = _sc_gather(a2, b2, dstc[c], srcc[c])
        u = _edge_mlp(upd1c[c], g2, None,
                      w1b[:, 512:].T, p1["edge_mlp"]["l2"]["W"].T,
                      p1["edge_mlp"]["l2"]["b"][None],
                      p1["edge_mlp"]["g"][None],
                      p1["edge_mlp"]["beta"][None])
        part2s.append(_sc_scatter(u, src3dc[c], n_pad))
    part2 = jnp.concatenate(part2s, axis=0)

    nw1 = p1["node_mlp"]["l1"]["W"]
    nodew1 = (nw1[:, :128].T, nw1[:, 128:256].T, nw1[:, 256:384].T,
              jnp.pad(nw1[:, 384:].T, ((0, 125), (0, 0))),
              p1["node_mlp"]["l1"]["b"][None],
              p1["node_mlp"]["l2"]["W"].T, p1["node_mlp"]["l2"]["b"][None],
              p1["node_mlp"]["g"][None], p1["node_mlp"]["beta"][None])
    wd1 = params["dec_l1"]["W"]
    wd2t = jnp.pad(params["dec_l2"]["W"].T, ((0, 0), (0, 125)))
    bd2 = jnp.pad(params["dec_l2"]["b"], (0, 125))[None]
    out_pad = _node_mlp(
        xi1, p_tab, part2, f_pad, nodew1,
        (r_tab, wd1[:, :128].T, wd1[:, 128:].T, params["dec_l1"]["b"][None],
         wd2t, bd2), "dec")
    return out_pad[:, :3]


# speedup vs baseline: 1.0022x; 1.0022x over previous
import functools
import math

import jax
import jax.numpy as jnp
from jax import lax
from jax.experimental import pallas as pl
from jax.experimental.pallas import tpu as pltpu
from jax.experimental.pallas import tpu_sc as plsc

F32 = jnp.float32
_EPS = 1e-5
_NC, _NS = 2, 16
_NW = _NC * _NS
_CE = 80
_NCH = 2
_NB = 1000
_EB = 4096


def _ln(h, g, beta):
    mu = jnp.mean(h, axis=-1, keepdims=True)
    var = jnp.mean((h - mu) ** 2, axis=-1, keepdims=True)
    return (h - mu) * lax.rsqrt(var + _EPS) * g + beta


def _full(shape):
    return pl.BlockSpec(shape, lambda i: tuple(0 for _ in shape))


def _enc_apply(xp, w1t, b1, w2t, b2, g, be, nb):
    r, din = xp.shape

    def body(x_ref, w1_ref, b1_ref, w2_ref, b2_ref, g_ref, be_ref, o_ref):
        h = jnp.maximum(
            jnp.dot(x_ref[...], w1_ref[...], preferred_element_type=F32)
            + b1_ref[...], 0.0)
        h = jnp.dot(h, w2_ref[...], preferred_element_type=F32) + b2_ref[...]
        o_ref[...] = _ln(h, g_ref[...], be_ref[...])

    return pl.pallas_call(
        body, grid=(r // nb,),
        in_specs=[pl.BlockSpec((nb, din), lambda i: (i, 0)),
                  _full(w1t.shape), _full(b1.shape), _full(w2t.shape),
                  _full(b2.shape), _full(g.shape), _full(be.shape)],
        out_specs=pl.BlockSpec((nb, 128), lambda i: (i, 0)),
        out_shape=jax.ShapeDtypeStruct((r, 128), F32),
    )(xp, w1t, b1, w2t, b2, g, be)


def _vt_x(v, xe):
    n = v.shape[0]

    def body(v_ref, x_ref, o_ref):
        @pl.when(pl.program_id(0) == 0)
        def _():
            o_ref[...] = jnp.zeros_like(o_ref)
        o_ref[...] += lax.dot_general(
            v_ref[...], x_ref[...], (((0,), (0,)), ((), ())),
            preferred_element_type=F32)

    return pl.pallas_call(
        body, grid=(n // _NB,),
        in_specs=[pl.BlockSpec((_NB, 128), lambda i: (i, 0)),
                  pl.BlockSpec((_NB, 128), lambda i: (i, 0))],
        out_specs=pl.BlockSpec((128, 128), lambda i: (0, 0)),
        out_shape=jax.ShapeDtypeStruct((128, 128), F32),
        compiler_params=pltpu.CompilerParams(
            dimension_semantics=("arbitrary",)),
    )(v, xe)


def _v_s(v, s):
    n = v.shape[0]

    def body(v_ref, s_ref, o_ref):
        o_ref[...] = jnp.dot(v_ref[...], s_ref[...],
                             preferred_element_type=F32)

    return pl.pallas_call(
        body, grid=(n // _NB,),
        in_specs=[pl.BlockSpec((_NB, 128), lambda i: (i, 0)), _full((128, 128))],
        out_specs=pl.BlockSpec((_NB, 128), lambda i: (i, 0)),
        out_shape=jax.ShapeDtypeStruct((n, 128), F32),
    )(v, s)


def _ab_tables(xs, p, wxa, wpa, ba, wxb, wpb):
    n = xs.shape[0]

    def body(x_ref, p_ref, wxa_r, wpa_r, ba_r, wxb_r, wpb_r, a_ref, b_ref):
        x = x_ref[...]
        pp = p_ref[...]
        a_ref[...] = (
            jnp.dot(x, wxa_r[...], preferred_element_type=F32)
            + jnp.dot(pp, wpa_r[...], preferred_element_type=F32)
            + ba_r[...])
        b_ref[...] = (
            jnp.dot(x, wxb_r[...], preferred_element_type=F32)
            + jnp.dot(pp, wpb_r[...], preferred_element_type=F32))

    blk = pl.BlockSpec((_NB, 128), lambda i: (i, 0))
    return pl.pallas_call(
        body, grid=(n // _NB,),
        in_specs=[blk, blk, _full((128, 128)), _full((128, 128)),
                  _full((1, 128)), _full((128, 128)), _full((128, 128))],
        out_specs=[blk, blk],
        out_shape=[jax.ShapeDtypeStruct((n, 128), F32)] * 2,
    )(xs, p, wxa, wpa, ba, wxb, wpb)


def _spc_layer1(sx1, sea_raw, mdst, msrc, msrct, ew, pw):
    def body(sx_ref, sraw_ref, mdst_ref, msrc_ref, msrct_ref,
             ew1t, eb1, ew2t, eb2, eg, ebe,
             wit, wjt, wet, pb1, pw2t, pb2, pg, pbe,
             wnxt, wnat, bn, wn2t, bn2, gn, ben,
             o_ref):
        sx = sx_ref[...]
        sea = jnp.maximum(
            jnp.dot(sraw_ref[...], ew1t[...], preferred_element_type=F32)
            + eb1[...], 0.0)
        sea = jnp.dot(sea, ew2t[...], preferred_element_type=F32) + eb2[...]
        sea = _ln(sea, eg[...], ebe[...])
        xiw = jnp.dot(sx, wit[...], preferred_element_type=F32)
        xjw = jnp.dot(sx, wjt[...], preferred_element_type=F32)
        pre = (jnp.dot(mdst_ref[...], xiw, preferred_element_type=F32)
               + jnp.dot(msrc_ref[...], xjw, preferred_element_type=F32)
               + jnp.dot(sea, wet[...], preferred_element_type=F32)
               + pb1[...])
        h = jnp.maximum(pre, 0.0)
        h = jnp.dot(h, pw2t[...], preferred_element_type=F32) + pb2[...]
        upd = _ln(h, pg[...], pbe[...]) + sea
        agg = jnp.dot(msrct_ref[...], upd, preferred_element_type=F32)
        pre_n = (jnp.dot(sx, wnxt[...], preferred_element_type=F32)
                 + jnp.dot(agg, wnat[...], preferred_element_type=F32)
                 + bn[...])
        hn = jnp.maximum(pre_n, 0.0)
        hn = jnp.dot(hn, wn2t[...], preferred_element_type=F32) + bn2[...]
        o_ref[...] = sx[:, :128] + _ln(hn, gn[...], ben[...])

    args = (sx1, sea_raw, mdst, msrc, msrct) + ew + pw
    return pl.pallas_call(
        body, out_shape=jax.ShapeDtypeStruct((128, 128), F32),
    )(*args)


def _edge_mlp(ea_in, g, wenc, wet, w2t, b2, gg, be):
    e = ea_in.shape[0]

    def body(*refs):
        if wenc is not None:
            (ea_ref, g_ref, e1t, eb1, e2t, eb2, eg, ebe,
             wet_r, w2t_r, b2_r, g_r, be_r, o_ref) = refs
            ea = jnp.maximum(
                jnp.dot(ea_ref[...], e1t[...], preferred_element_type=F32)
                + eb1[...], 0.0)
            ea = jnp.dot(ea, e2t[...], preferred_element_type=F32) + eb2[...]
            ea = _ln(ea, eg[...], ebe[...])
        else:
            (ea_ref, g_ref, wet_r, w2t_r, b2_r, g_r, be_r, o_ref) = refs
            ea = ea_ref[...]
        pre = (g_ref[...]
               + jnp.dot(ea, wet_r[...], preferred_element_type=F32))
        h = jnp.maximum(pre, 0.0)
        h = jnp.dot(h, w2t_r[...], preferred_element_type=F32) + b2_r[...]
        o_ref[...] = _ln(h, g_r[...], be_r[...]) + ea

    w = ea_in.shape[1]
    in_specs = [pl.BlockSpec((_EB, w), lambda i: (i, 0)),
                pl.BlockSpec((_EB, 128), lambda i: (i, 0))]
    args = [ea_in, g]
    if wenc is not None:
        for a in wenc:
            in_specs.append(_full(a.shape))
            args.append(a)
    for a in (wet, w2t, b2, gg, be):
        in_specs.append(_full(a.shape))
        args.append(a)
    return pl.pallas_call(
        body, grid=(e // _EB,),
        in_specs=in_specs,
        out_specs=pl.BlockSpec((_EB, 128), lambda i: (i, 0)),
        out_shape=jax.ShapeDtypeStruct((e, 128), F32),
    )(*args)


def _node_mlp(xs, p, parts, f, nw, extra, mode):
    n = xs.shape[0]
    np_parts = parts.shape[0]
    wnxt, wnpt, wnat, wnft, bn, wn2t, bn2, gn, ben = nw

    def body(*refs):
        xs_ref, p_ref = refs[0], refs[1]
        prefs = refs[2:2 + np_parts]
        f_ref = refs[2 + np_parts]
        rest = refs[3 + np_parts:]
        if mode == "dec":
            r_ref = rest[0]
            rest = rest[1:]
        (wnxt_r, wnpt_r, wnat_r, wnft_r, bn_r,
         wn2t_r, bn2_r, gn_r, ben_r) = rest[:9]
        wext = rest[9:14]
        outs = rest[14:]
        xsv = xs_ref[...]
        pv = p_ref[...]
        agg = prefs[0][0]
        for pr in prefs[1:]:
            agg = agg + pr[0]
        pre = (jnp.dot(xsv, wnxt_r[...], preferred_element_type=F32)
               + jnp.dot(pv, wnpt_r[...], preferred_element_type=F32)
               + jnp.dot(agg, wnat_r[...], preferred_element_type=F32)
               + jnp.dot(f_ref[...], wnft_r[...], preferred_element_type=F32)
               + bn_r[...])
        h = jnp.maximum(pre, 0.0)
        h = jnp.dot(h, wn2t_r[...], preferred_element_type=F32) + bn2_r[...]
        xi = xsv + _ln(h, gn_r[...], ben_r[...])
        if mode == "ab":
            wxa, wpa, ba, wxb, wpb = wext
            xi_ref, a_ref, b_ref = outs
            xi_ref[...] = xi
            a_ref[...] = (
                jnp.dot(xi, wxa[...], preferred_element_type=F32)
                + jnp.dot(pv, wpa[...], preferred_element_type=F32)
                + ba[...])
            b_ref[...] = (
                jnp.dot(xi, wxb[...], preferred_element_type=F32)
                + jnp.dot(pv, wpb[...], preferred_element_type=F32))
        else:
            wd1xt, wd1rt, bd1, wd2t, bd2 = wext
            o_ref, = outs
            hd = jnp.maximum(
                jnp.dot(xi, wd1xt[...], preferred_element_type=F32)
                + jnp.dot(r_ref[...], wd1rt[...], preferred_element_type=F32)
                + bd1[...], 0.0)
            o_ref[...] = (jnp.dot(hd, wd2t[...], preferred_element_type=F32)
                          + bd2[...])

    blk = pl.BlockSpec((_NB, 128), lambda i: (i, 0))
    args = [xs, p] + [parts] * np_parts + [f]
    in_specs = [blk, blk]
    for j in range(np_parts):
        in_specs.append(
            pl.BlockSpec((1, _NB, 128),
                         functools.partial(lambda i, jj: (jj, i, 0), jj=j)))
    in_specs.append(blk)
    if mode == "dec":
        args.append(extra[0])
        in_specs.append(blk)
        wlist = nw + tuple(extra[1:])
    else:
        wlist = nw + tuple(extra)
    for a in wlist:
        args.append(a)
        in_specs.append(_full(a.shape))
    if mode == "ab":
        out_specs = [blk, blk, blk]
        out_shape = [jax.ShapeDtypeStruct((n, 128), F32)] * 3
    else:
        out_specs = blk
        out_shape = jax.ShapeDtypeStruct((n, 128), F32)
    return pl.pallas_call(
        body, grid=(n // _NB,),
        in_specs=in_specs, out_specs=out_specs, out_shape=out_shape,
    )(*args)


def _sc_gather(a_tab, b_tab, dst, src):
    e = dst.shape[0]
    per_w = e // _NW
    nck = per_w // _CE
    mesh = plsc.VectorSubcoreMesh(core_axis_name="c", subcore_axis_name="s")

    @functools.partial(
        pl.kernel, mesh=mesh,
        out_type=jax.ShapeDtypeStruct((e, 128), F32),
        scratch_types=[
            pltpu.VMEM((per_w,), jnp.int32),
            pltpu.VMEM((per_w,), jnp.int32),
            pltpu.VMEM((2, _CE, 128), F32),
            pltpu.VMEM((2, _CE, 128), F32),
            pltpu.SemaphoreType.DMA,
            pltpu.SemaphoreType.DMA,
            pltpu.SemaphoreType.DMA,
            pltpu.SemaphoreType.DMA,
        ],
    )
    def k(a_h, b_h, dst_h, src_h, o_h,
          idxd, idxs, bufa, bufb, sg0, sg1, sw0, sw1):
        wid = lax.axis_index("s") * _NC + lax.axis_index("c")
        base = wid * per_w
        pltpu.sync_copy(dst_h.at[pl.ds(base, per_w)], idxd)
        pltpu.sync_copy(src_h.at[pl.ds(base, per_w)], idxs)
        sgs = (sg0, sg1)
        sws = (sw0, sw1)

        def issue(ci, p):
            off = ci * _CE
            pltpu.async_copy(a_h.at[idxd.at[pl.ds(off, _CE)]], bufa.at[p],
                             sgs[p])
            pltpu.async_copy(b_h.at[idxs.at[pl.ds(off, _CE)]], bufb.at[p],
                             sgs[p])

        def wait_g(p):
            pltpu.make_async_copy(a_h.at[idxd.at[pl.ds(0, _CE)]], bufa.at[p],
                                  sgs[p]).wait()
            pltpu.make_async_copy(b_h.at[idxs.at[pl.ds(0, _CE)]], bufb.at[p],
                                  sgs[p]).wait()

        def add(p):
            def row(r, carry):
                for j in range(8):
                    sl = pl.ds(j * 16, 16)
                    bufa[p, r, sl] = bufa[p, r, sl] + bufb[p, r, sl]
                return carry

            lax.fori_loop(0, _CE, row, 0)

        def write(ci, p):
            off = base + ci * _CE
            pltpu.async_copy(bufa.at[p], o_h.at[pl.ds(off, _CE)], sws[p])

        def wait_w(p):
            pltpu.make_async_copy(bufa.at[p], o_h.at[pl.ds(base, _CE)],
                                  sws[p]).wait()

        issue(0, 0)
        issue(1, 1)

        def step(ci, carry):
            wait_g(0)
            add(0)
            write(ci, 0)
            wait_g(1)
            add(1)
            write(ci + 1, 1)
            wait_w(0)
            issue(ci + 2, 0)
            wait_w(1)

            @pl.when(ci + 3 < nck)
            def _():
                issue(ci + 3, 1)

            return carry

        if nck % 2:
            lax.fori_loop(0, (nck - 1) // 2, lambda i, c: step(2 * i, c), 0)
            wait_g(0)
            add(0)
            write(nck - 1, 0)
            wait_w(0)
        else:
            lax.fori_loop(0, (nck - 2) // 2, lambda i, c: step(2 * i, c), 0)
            wait_g(0)
            add(0)
            write(nck - 2, 0)
            wait_g(1)
            add(1)
            write(nck - 1, 1)
            wait_w(0)
            wait_w(1)

    return k(a_tab, b_tab, dst, src)


def _sc_scatter(upd, src3d, n_pad):
    rows_w = src3d.shape[1]
    rps = n_pad // _NS
    zr = 32
    mesh = plsc.VectorSubcoreMesh(core_axis_name="c", subcore_axis_name="s")

    @functools.partial(
        pl.kernel, mesh=mesh,
        out_type=jax.ShapeDtypeStruct((_NC, n_pad, 128), F32),
        scratch_types=[
            pltpu.VMEM((rows_w, _CE), jnp.int32),
            pltpu.VMEM((2, _CE, 128), F32),
            pltpu.VMEM((zr, 128), F32),
            pltpu.VMEM_SHARED((n_pad, 128), F32),
            pltpu.SemaphoreType.DMA,
            pltpu.SemaphoreType.DMA,
            pltpu.SemaphoreType.DMA,
            pltpu.SemaphoreType.DMA,
        ],
    )
    def k(upd_h, src_h, out_h, idx2d, rows, stage, acc, si0, si1, ss0, ss1):
        c = lax.axis_index("c")
        s = lax.axis_index("s")
        wid = s * _NC + c
        sis = (si0, si1)
        sss = (ss0, ss1)

        def zrow(r, carry):
            for j in range(8):
                stage[r, pl.ds(j * 16, 16)] = jnp.zeros((16,), F32)
            return carry

        lax.fori_loop(0, zr, zrow, 0)

        def zcopy(t, carry):
            pltpu.sync_copy(stage, acc.at[pl.ds(s * rps + t * zr, zr)])
            return carry

        lax.fori_loop(0, rps // zr, zcopy, 0)
        pltpu.sync_copy(src_h.at[wid], idx2d)
        plsc.subcore_barrier()

        base = wid * rows_w * _CE

        def issue_in(ci, p):
            pltpu.async_copy(upd_h.at[pl.ds(base + ci * _CE, _CE)],
                             rows.at[p], sis[p])

        def wait_in(p):
            pltpu.make_async_copy(upd_h.at[pl.ds(base, _CE)], rows.at[p],
                                  sis[p]).wait()

        def issue_sc(ci, p):
            pltpu.async_copy(rows.at[p], acc.at[idx2d.at[ci]], sss[p],
                             add=True)

        def wait_sc(ci, p):
            pltpu.make_async_copy(rows.at[p], acc.at[idx2d.at[ci]],
                                  sss[p]).wait()

        issue_in(0, 0)
        issue_in(1, 1)

        def step(ci, carry):
            wait_in(0)
            issue_sc(ci, 0)
            wait_in(1)
            issue_sc(ci + 1, 1)
            wait_sc(ci, 0)
            issue_in(ci + 2, 0)
            wait_sc(ci + 1, 1)

            @pl.when(ci + 3 < rows_w)
            def _():
                issue_in(ci + 3, 1)

            return carry

        if rows_w % 2:
            lax.fori_loop(0, (rows_w - 1) // 2,
                          lambda i, cr: step(2 * i, cr), 0)
            wait_in(0)
            issue_sc(rows_w - 1, 0)
            wait_sc(rows_w - 1, 0)
        else:
            lax.fori_loop(0, (rows_w - 2) // 2,
                          lambda i, cr: step(2 * i, cr), 0)
            wait_in(0)
            issue_sc(rows_w - 2, 0)
            wait_in(1)
            issue_sc(rows_w - 1, 1)
            wait_sc(rows_w - 2, 0)
            wait_sc(rows_w - 1, 1)
        plsc.subcore_barrier()

        def wback(t, carry):
            r0 = s * rps + t * zr
            pltpu.sync_copy(acc.at[pl.ds(r0, zr)], stage)
            pltpu.sync_copy(stage, out_h.at[c, pl.ds(r0, zr)])
            return carry

        lax.fori_loop(0, rps // zr, wback, 0)

    return k(upd, src3d)


def _enc_w(p, pad_to=None):
    w1t = p["l1"]["W"].T
    if pad_to is not None:
        w1t = jnp.pad(w1t, ((0, pad_to - w1t.shape[0]), (0, 0)))
    return (w1t, p["l1"]["b"][None], p["l2"]["W"].T, p["l2"]["b"][None],
            p["g"][None], p["beta"][None])


def kernel(x, edge_attr, del_ext_force, spc_x, spc_edge_attr, V, params,
           edge_index, spc_edge_index):
    n = x.shape[0]
    n_pad = ((n + 16 * 128 - 1) // (16 * 128)) * (16 * 128)
    k_dim = spc_x.shape[0]
    e = edge_index.shape[1]
    align = (_NW * _CE) * _EB // math.gcd(_NW * _CE, _EB)
    ec = ((e // _NCH + align - 1) // align) * align
    nworker = _NCH * _NW
    per_real = e // nworker
    per_w = ec // _NW
    padw = per_w - per_real
    idx = edge_index.astype(jnp.int32)
    dummy_src = jnp.broadcast_to(
        n + (jnp.arange(padw, dtype=jnp.int32) % (n_pad - n)),
        (nworker, padw))
    srcp = jnp.concatenate(
        [idx[0].reshape(nworker, per_real), dummy_src], axis=1).reshape(-1)
    dstp = jnp.concatenate(
        [idx[1].reshape(nworker, per_real),
         jnp.zeros((nworker, padw), jnp.int32)], axis=1).reshape(-1)
    ea_p = jnp.concatenate(
        [edge_attr.reshape(nworker, per_real, edge_attr.shape[1]),
         jnp.zeros((nworker, padw, edge_attr.shape[1]), edge_attr.dtype)],
        axis=1).reshape(_NCH * ec, edge_attr.shape[1])
    srcc = [srcp[c * ec:(c + 1) * ec] for c in range(_NCH)]
    dstc = [dstp[c * ec:(c + 1) * ec] for c in range(_NCH)]
    src3dc = [s.reshape(_NW, -1, _CE) for s in srcc]

    f_pad = jnp.pad(del_ext_force, ((0, 0), (0, 128 - del_ext_force.shape[1])))
    spc_x_pad = jnp.pad(spc_x, ((0, 0), (0, 128 - spc_x.shape[1])))
    sea_raw = jnp.pad(spc_edge_attr, ((0, 0), (0, 128 - spc_edge_attr.shape[1])))

    kk = jnp.arange(k_dim, dtype=jnp.int32)
    mdst = (spc_edge_index[1][:, None] == kk[None, :]).astype(F32)
    msrc = (spc_edge_index[0][:, None] == kk[None, :]).astype(F32)
    msrct = (kk[:, None] == spc_edge_index[0][None, :]).astype(F32)

    xe = _enc_apply(x, *_enc_w(params["node_encoder"]), _NB)
    se = _enc_apply(spc_x_pad, *_enc_w(params["spc_node_encoder"], 128), 128)

    q = _vt_x(V, xe)
    p_tab = _v_s(V, se)
    sx1 = jnp.concatenate([se, q], axis=1)

    sp = params["spc_proc"][0]
    sw1 = sp["edge_mlp"]["l1"]["W"]
    snw1 = sp["node_mlp"]["l1"]["W"]
    sxi1 = _spc_layer1(
        sx1, sea_raw, mdst, msrc, msrct,
        _enc_w(params["spc_edge_encoder"], 128),
        (sw1[:, :256].T, sw1[:, 256:512].T, sw1[:, 512:].T,
         sp["edge_mlp"]["l1"]["b"][None],
         sp["edge_mlp"]["l2"]["W"].T, sp["edge_mlp"]["l2"]["b"][None],
         sp["edge_mlp"]["g"][None], sp["edge_mlp"]["beta"][None],
         snw1[:, :256].T, snw1[:, 256:].T, sp["node_mlp"]["l1"]["b"][None],
         sp["node_mlp"]["l2"]["W"].T, sp["node_mlp"]["l2"]["b"][None],
         sp["node_mlp"]["g"][None], sp["node_mlp"]["beta"][None]))
    r_tab = _v_s(V, sxi1)

    p0 = params["proc"][0]
    w1 = p0["edge_mlp"]["l1"]["W"]
    a1, b1 = _ab_tables(xe, p_tab, w1[:, :128].T, w1[:, 128:256].T,
                        p0["edge_mlp"]["l1"]["b"][None],
                        w1[:, 256:384].T, w1[:, 384:512].T)
    upd1c = []
    part1s = []
    for c in range(_NCH):
        g1 = _sc_gather(a1, b1, dstc[c], srcc[c])
        u = _edge_mlp(ea_p[c * ec:(c + 1) * ec], g1,
                      _enc_w(params["edge_encoder"]),
                      w1[:, 512:].T, p0["edge_mlp"]["l2"]["W"].T,
                      p0["edge_mlp"]["l2"]["b"][None],
                      p0["edge_mlp"]["g"][None],
                      p0["edge_mlp"]["beta"][None])
        upd1c.append(u)
        part1s.append(_sc_scatter(u, src3dc[c], n_pad))
    part1 = jnp.concatenate(part1s, axis=0)

    p1 = params["proc"][1]
    w1b = p1["edge_mlp"]["l1"]["W"]
    nw0 = p0["node_mlp"]["l1"]["W"]
    nodew0 = (nw0[:, :128].T, nw0[:, 128:256].T, nw0[:, 256:384].T,
              jnp.pad(nw0[:, 384:].T, ((0, 125), (0, 0))),
              p0["node_mlp"]["l1"]["b"][None],
              p0["node_mlp"]["l2"]["W"].T, p0["node_mlp"]["l2"]["b"][None],
              p0["node_mlp"]["g"][None], p0["node_mlp"]["beta"][None])
    xi1, a2, b2 = _node_mlp(
        xe, p_tab, part1, f_pad, nodew0,
        (w1b[:, :128].T, w1b[:, 128:256].T, p1["edge_mlp"]["l1"]["b"][None],
         w1b[:, 256:384].T, w1b[:, 384:512].T), "ab")

    part2s = []
    for c in range(_NCH):
        g2 = _sc_gather(a2, b2, dstc[c], srcc[c])
        u = _edge_mlp(upd1c[c], g2, None,
                      w1b[:, 512:].T, p1["edge_mlp"]["l2"]["W"].T,
                      p1["edge_mlp"]["l2"]["b"][None],
                      p1["edge_mlp"]["g"][None],
                      p1["edge_mlp"]["beta"][None])
        part2s.append(_sc_scatter(u, src3dc[c], n_pad))
    part2 = jnp.concatenate(part2s, axis=0)

    nw1 = p1["node_mlp"]["l1"]["W"]
    nodew1 = (nw1[:, :128].T, nw1[:, 128:256].T, nw1[:, 256:384].T,
              jnp.pad(nw1[:, 384:].T, ((0, 125), (0, 0))),
              p1["node_mlp"]["l1"]["b"][None],
              p1["node_mlp"]["l2"]["W"].T, p1["node_mlp"]["l2"]["b"][None],
              p1["node_mlp"]["g"][None], p1["node_mlp"]["beta"][None])
    wd1 = params["dec_l1"]["W"]
    wd2t = jnp.pad(params["dec_l2"]["W"].T, ((0, 0), (0, 125)))
    bd2 = jnp.pad(params["dec_l2"]["b"], (0, 125))[None]
    out_pad = _node_mlp(
        xi1, p_tab, part2, f_pad, nodew1,
        (r_tab, wd1[:, :128].T, wd1[:, 128:].T, params["dec_l1"]["b"][None],
         wd2t, bd2), "dec")
    return out_pad[:, :3]


# speedup vs baseline: 1.5114x; 1.5081x over previous
import functools

import jax
import jax.numpy as jnp
from jax import lax
from jax.experimental import pallas as pl
from jax.experimental.pallas import tpu as pltpu
from jax.experimental.pallas import tpu_sc as plsc

F32 = jnp.float32
_EPS = 1e-5
_NC, _NS = 2, 16
_NW = _NC * _NS
_CE = 40
_NCH = 2
_NB = 1000
_EB = 3200


def _ln(h, g, beta):
    mu = jnp.mean(h, axis=-1, keepdims=True)
    var = jnp.mean((h - mu) ** 2, axis=-1, keepdims=True)
    return (h - mu) * lax.rsqrt(var + _EPS) * g + beta


def _full(shape):
    return pl.BlockSpec(shape, lambda i: tuple(0 for _ in shape))


def _enc_apply(xp, w1t, b1, w2t, b2, g, be, nb):
    r, din = xp.shape

    def body(x_ref, w1_ref, b1_ref, w2_ref, b2_ref, g_ref, be_ref, o_ref):
        h = jnp.maximum(
            jnp.dot(x_ref[...], w1_ref[...], preferred_element_type=F32)
            + b1_ref[...], 0.0)
        h = jnp.dot(h, w2_ref[...], preferred_element_type=F32) + b2_ref[...]
        o_ref[...] = _ln(h, g_ref[...], be_ref[...])

    return pl.pallas_call(
        body, grid=(r // nb,),
        in_specs=[pl.BlockSpec((nb, din), lambda i: (i, 0)),
                  _full(w1t.shape), _full(b1.shape), _full(w2t.shape),
                  _full(b2.shape), _full(g.shape), _full(be.shape)],
        out_specs=pl.BlockSpec((nb, 128), lambda i: (i, 0)),
        out_shape=jax.ShapeDtypeStruct((r, 128), F32),
    )(xp, w1t, b1, w2t, b2, g, be)


def _vt_x(v, xe):
    n = v.shape[0]

    def body(v_ref, x_ref, o_ref):
        @pl.when(pl.program_id(0) == 0)
        def _():
            o_ref[...] = jnp.zeros_like(o_ref)
        o_ref[...] += lax.dot_general(
            v_ref[...], x_ref[...], (((0,), (0,)), ((), ())),
            preferred_element_type=F32)

    return pl.pallas_call(
        body, grid=(n // _NB,),
        in_specs=[pl.BlockSpec((_NB, 128), lambda i: (i, 0)),
                  pl.BlockSpec((_NB, 128), lambda i: (i, 0))],
        out_specs=pl.BlockSpec((128, 128), lambda i: (0, 0)),
        out_shape=jax.ShapeDtypeStruct((128, 128), F32),
        compiler_params=pltpu.CompilerParams(
            dimension_semantics=("arbitrary",)),
    )(v, xe)


def _v_s(v, s):
    n = v.shape[0]

    def body(v_ref, s_ref, o_ref):
        o_ref[...] = jnp.dot(v_ref[...], s_ref[...],
                             preferred_element_type=F32)

    return pl.pallas_call(
        body, grid=(n // _NB,),
        in_specs=[pl.BlockSpec((_NB, 128), lambda i: (i, 0)), _full((128, 128))],
        out_specs=pl.BlockSpec((_NB, 128), lambda i: (i, 0)),
        out_shape=jax.ShapeDtypeStruct((n, 128), F32),
    )(v, s)


def _ab_tables(xs, p, wxa, wpa, ba, wxb, wpb):
    n = xs.shape[0]

    def body(x_ref, p_ref, wxa_r, wpa_r, ba_r, wxb_r, wpb_r, a_ref, b_ref):
        x = x_ref[...]
        pp = p_ref[...]
        a_ref[...] = (
            jnp.dot(x, wxa_r[...], preferred_element_type=F32)
            + jnp.dot(pp, wpa_r[...], preferred_element_type=F32)
            + ba_r[...])
        b_ref[...] = (
            jnp.dot(x, wxb_r[...], preferred_element_type=F32)
            + jnp.dot(pp, wpb_r[...], preferred_element_type=F32))

    blk = pl.BlockSpec((_NB, 128), lambda i: (i, 0))
    return pl.pallas_call(
        body, grid=(n // _NB,),
        in_specs=[blk, blk, _full((128, 128)), _full((128, 128)),
                  _full((1, 128)), _full((128, 128)), _full((128, 128))],
        out_specs=[blk, blk],
        out_shape=[jax.ShapeDtypeStruct((n, 128), F32)] * 2,
    )(xs, p, wxa, wpa, ba, wxb, wpb)


def _spc_layer1(sx1, sea_raw, mdst, msrc, msrct, ew, pw):
    def body(sx_ref, sraw_ref, mdst_ref, msrc_ref, msrct_ref,
             ew1t, eb1, ew2t, eb2, eg, ebe,
             wit, wjt, wet, pb1, pw2t, pb2, pg, pbe,
             wnxt, wnat, bn, wn2t, bn2, gn, ben,
             o_ref):
        sx = sx_ref[...]
        sea = jnp.maximum(
            jnp.dot(sraw_ref[...], ew1t[...], preferred_element_type=F32)
            + eb1[...], 0.0)
        sea = jnp.dot(sea, ew2t[...], preferred_element_type=F32) + eb2[...]
        sea = _ln(sea, eg[...], ebe[...])
        xiw = jnp.dot(sx, wit[...], preferred_element_type=F32)
        xjw = jnp.dot(sx, wjt[...], preferred_element_type=F32)
        pre = (jnp.dot(mdst_ref[...], xiw, preferred_element_type=F32)
               + jnp.dot(msrc_ref[...], xjw, preferred_element_type=F32)
               + jnp.dot(sea, wet[...], preferred_element_type=F32)
               + pb1[...])
        h = jnp.maximum(pre, 0.0)
        h = jnp.dot(h, pw2t[...], preferred_element_type=F32) + pb2[...]
        upd = _ln(h, pg[...], pbe[...]) + sea
        agg = jnp.dot(msrct_ref[...], upd, preferred_element_type=F32)
        pre_n = (jnp.dot(sx, wnxt[...], preferred_element_type=F32)
                 + jnp.dot(agg, wnat[...], preferred_element_type=F32)
                 + bn[...])
        hn = jnp.maximum(pre_n, 0.0)
        hn = jnp.dot(hn, wn2t[...], preferred_element_type=F32) + bn2[...]
        o_ref[...] = sx[:, :128] + _ln(hn, gn[...], ben[...])

    args = (sx1, sea_raw, mdst, msrc, msrct) + ew + pw
    return pl.pallas_call(
        body, out_shape=jax.ShapeDtypeStruct((128, 128), F32),
    )(*args)


def _edge_mlp(ea_in, g, wenc, wet, w2t, b2, gg, be):
    e = ea_in.shape[0]

    def body(*refs):
        if wenc is not None:
            (ea_ref, g_ref, e1t, eb1, e2t, eb2, eg, ebe,
             wet_r, w2t_r, b2_r, g_r, be_r, o_ref) = refs
            ea = jnp.maximum(
                jnp.dot(ea_ref[...], e1t[...], preferred_element_type=F32)
                + eb1[...], 0.0)
            ea = jnp.dot(ea, e2t[...], preferred_element_type=F32) + eb2[...]
            ea = _ln(ea, eg[...], ebe[...])
        else:
            (ea_ref, g_ref, wet_r, w2t_r, b2_r, g_r, be_r, o_ref) = refs
            ea = ea_ref[...]
        pre = (g_ref[...]
               + jnp.dot(ea, wet_r[...], preferred_element_type=F32))
        h = jnp.maximum(pre, 0.0)
        h = jnp.dot(h, w2t_r[...], preferred_element_type=F32) + b2_r[...]
        o_ref[...] = _ln(h, g_r[...], be_r[...]) + ea

    w = ea_in.shape[1]
    in_specs = [pl.BlockSpec((_EB, w), lambda i: (i, 0)),
                pl.BlockSpec((_EB, 128), lambda i: (i, 0))]
    args = [ea_in, g]
    if wenc is not None:
        for a in wenc:
            in_specs.append(_full(a.shape))
            args.append(a)
    for a in (wet, w2t, b2, gg, be):
        in_specs.append(_full(a.shape))
        args.append(a)
    return pl.pallas_call(
        body, grid=(e // _EB,),
        in_specs=in_specs,
        out_specs=pl.BlockSpec((_EB, 128), lambda i: (i, 0)),
        out_shape=jax.ShapeDtypeStruct((e, 128), F32),
    )(*args)


def _node_mlp(xs, p, parts, f, nw, extra, mode):
    n = xs.shape[0]
    np_parts = parts.shape[0]
    wnxt, wnpt, wnat, wnft, bn, wn2t, bn2, gn, ben = nw

    def body(*refs):
        xs_ref, p_ref = refs[0], refs[1]
        prefs = refs[2:2 + np_parts]
        f_ref = refs[2 + np_parts]
        rest = refs[3 + np_parts:]
        if mode == "dec":
            r_ref = rest[0]
            rest = rest[1:]
        (wnxt_r, wnpt_r, wnat_r, wnft_r, bn_r,
         wn2t_r, bn2_r, gn_r, ben_r) = rest[:9]
        wext = rest[9:14]
        outs = rest[14:]
        xsv = xs_ref[...]
        pv = p_ref[...]
        agg = prefs[0][0]
        for pr in prefs[1:]:
            agg = agg + pr[0]
        pre = (jnp.dot(xsv, wnxt_r[...], preferred_element_type=F32)
               + jnp.dot(pv, wnpt_r[...], preferred_element_type=F32)
               + jnp.dot(agg, wnat_r[...], preferred_element_type=F32)
               + jnp.dot(f_ref[...], wnft_r[...], preferred_element_type=F32)
               + bn_r[...])
        h = jnp.maximum(pre, 0.0)
        h = jnp.dot(h, wn2t_r[...], preferred_element_type=F32) + bn2_r[...]
        xi = xsv + _ln(h, gn_r[...], ben_r[...])
        if mode == "ab":
            wxa, wpa, ba, wxb, wpb = wext
            xi_ref, a_ref, b_ref = outs
            xi_ref[...] = xi
            a_ref[...] = (
                jnp.dot(xi, wxa[...], preferred_element_type=F32)
                + jnp.dot(pv, wpa[...], preferred_element_type=F32)
                + ba[...])
            b_ref[...] = (
                jnp.dot(xi, wxb[...], preferred_element_type=F32)
                + jnp.dot(pv, wpb[...], preferred_element_type=F32))
        else:
            wd1xt, wd1rt, bd1, wd2t, bd2 = wext
            o_ref, = outs
            hd = jnp.maximum(
                jnp.dot(xi, wd1xt[...], preferred_element_type=F32)
                + jnp.dot(r_ref[...], wd1rt[...], preferred_element_type=F32)
                + bd1[...], 0.0)
            o_ref[...] = (jnp.dot(hd, wd2t[...], preferred_element_type=F32)
                          + bd2[...])

    blk = pl.BlockSpec((_NB, 128), lambda i: (i, 0))
    args = [xs, p] + [parts] * np_parts + [f]
    in_specs = [blk, blk]
    for j in range(np_parts):
        in_specs.append(
            pl.BlockSpec((1, _NB, 128),
                         functools.partial(lambda i, jj: (jj, i, 0), jj=j)))
    in_specs.append(blk)
    if mode == "dec":
        args.append(extra[0])
        in_specs.append(blk)
        wlist = nw + tuple(extra[1:])
    else:
        wlist = nw + tuple(extra)
    for a in wlist:
        args.append(a)
        in_specs.append(_full(a.shape))
    if mode == "ab":
        out_specs = [blk, blk, blk]
        out_shape = [jax.ShapeDtypeStruct((n, 128), F32)] * 3
    else:
        out_specs = blk
        out_shape = jax.ShapeDtypeStruct((n, 128), F32)
    return pl.pallas_call(
        body, grid=(n // _NB,),
        in_specs=in_specs, out_specs=out_specs, out_shape=out_shape,
    )(*args)


def _sc_gather(a_tab, b_tab, dst, src):
    e = dst.shape[0]
    per_w = e // _NW
    nck = per_w // _CE
    mesh = plsc.VectorSubcoreMesh(core_axis_name="c", subcore_axis_name="s")

    @functools.partial(
        pl.kernel, mesh=mesh,
        out_type=jax.ShapeDtypeStruct((e, 128), F32),
        scratch_types=[
            pltpu.VMEM((per_w,), jnp.int32),
            pltpu.VMEM((per_w,), jnp.int32),
            pltpu.VMEM((2, _CE, 128), F32),
            pltpu.VMEM((2, _CE, 128), F32),
            pltpu.SemaphoreType.DMA,
            pltpu.SemaphoreType.DMA,
            pltpu.SemaphoreType.DMA,
            pltpu.SemaphoreType.DMA,
        ],
    )
    def k(a_h, b_h, dst_h, src_h, o_h,
          idxd, idxs, bufa, bufb, sg0, sg1, sw0, sw1):
        wid = lax.axis_index("s") * _NC + lax.axis_index("c")
        base = wid * per_w
        pltpu.sync_copy(dst_h.at[pl.ds(base, per_w)], idxd)
        pltpu.sync_copy(src_h.at[pl.ds(base, per_w)], idxs)
        sgs = (sg0, sg1)
        sws = (sw0, sw1)

        def issue(ci, p):
            off = ci * _CE
            pltpu.async_copy(a_h.at[idxd.at[pl.ds(off, _CE)]], bufa.at[p],
                             sgs[p])
            pltpu.async_copy(b_h.at[idxs.at[pl.ds(off, _CE)]], bufb.at[p],
                             sgs[p])

        def wait_g(p):
            pltpu.make_async_copy(a_h.at[idxd.at[pl.ds(0, _CE)]], bufa.at[p],
                                  sgs[p]).wait()
            pltpu.make_async_copy(b_h.at[idxs.at[pl.ds(0, _CE)]], bufb.at[p],
                                  sgs[p]).wait()

        def add(p):
            def row(r, carry):
                for j in range(8):
                    sl = pl.ds(j * 16, 16)
                    bufa[p, r, sl] = bufa[p, r, sl] + bufb[p, r, sl]
                return carry

            lax.fori_loop(0, _CE, row, 0)

        def write(ci, p):
            off = base + ci * _CE
            pltpu.async_copy(bufa.at[p], o_h.at[pl.ds(off, _CE)], sws[p])

        def wait_w(p):
            pltpu.make_async_copy(bufa.at[p], o_h.at[pl.ds(base, _CE)],
                                  sws[p]).wait()

        issue(0, 0)
        issue(1, 1)

        def step(ci, carry):
            wait_g(0)
            add(0)
            write(ci, 0)
            wait_g(1)
            add(1)
            write(ci + 1, 1)
            wait_w(0)
            issue(ci + 2, 0)
            wait_w(1)

            @pl.when(ci + 3 < nck)
            def _():
                issue(ci + 3, 1)

            return carry

        if nck % 2:
            lax.fori_loop(0, (nck - 1) // 2, lambda i, c: step(2 * i, c), 0)
            wait_g(0)
            add(0)
            write(nck - 1, 0)
            wait_w(0)
        else:
            lax.fori_loop(0, (nck - 2) // 2, lambda i, c: step(2 * i, c), 0)
            wait_g(0)
            add(0)
            write(nck - 2, 0)
            wait_g(1)
            add(1)
            write(nck - 1, 1)
            wait_w(0)
            wait_w(1)

    return k(a_tab, b_tab, dst, src)


def _sc_scatter(upd, src3d, n_pad):
    rows_w = src3d.shape[1]
    rps = n_pad // _NS
    zr = 64
    mesh = plsc.VectorSubcoreMesh(core_axis_name="c", subcore_axis_name="s")

    @functools.partial(
        pl.kernel, mesh=mesh,
        out_type=jax.ShapeDtypeStruct((_NC, n_pad, 128), F32),
        scratch_types=[
            pltpu.VMEM((rows_w, _CE), jnp.int32),
            pltpu.VMEM((2, _CE, 128), F32),
            pltpu.VMEM((zr, 128), F32),
            pltpu.VMEM_SHARED((n_pad, 128), F32),
            pltpu.SemaphoreType.DMA,
            pltpu.SemaphoreType.DMA,
            pltpu.SemaphoreType.DMA,
            pltpu.SemaphoreType.DMA,
        ],
    )
    def k(upd_h, src_h, out_h, idx2d, rows, stage, acc, si0, si1, ss0, ss1):
        c = lax.axis_index("c")
        s = lax.axis_index("s")
        wid = s * _NC + c
        sis = (si0, si1)
        sss = (ss0, ss1)

        def zrow(r, carry):
            for j in range(8):
                stage[r, pl.ds(j * 16, 16)] = jnp.zeros((16,), F32)
            return carry

        lax.fori_loop(0, zr, zrow, 0)

        def zcopy(t, carry):
            pltpu.sync_copy(stage, acc.at[pl.ds(s * rps + t * zr, zr)])
            return carry

        lax.fori_loop(0, rps // zr, zcopy, 0)
        pltpu.sync_copy(src_h.at[wid], idx2d)
        plsc.subcore_barrier()

        base = wid * rows_w * _CE

        def issue_in(ci, p):
            pltpu.async_copy(upd_h.at[pl.ds(base + ci * _CE, _CE)],
                             rows.at[p], sis[p])

        def wait_in(p):
            pltpu.make_async_copy(upd_h.at[pl.ds(base, _CE)], rows.at[p],
                                  sis[p]).wait()

        def issue_sc(ci, p):
            pltpu.async_copy(rows.at[p], acc.at[idx2d.at[ci]], sss[p],
                             add=True)

        def wait_sc(ci, p):
            pltpu.make_async_copy(rows.at[p], acc.at[idx2d.at[ci]],
                                  sss[p]).wait()

        issue_in(0, 0)
        issue_in(1, 1)

        def step(ci, carry):
            wait_in(0)
            issue_sc(ci, 0)
            wait_in(1)
            issue_sc(ci + 1, 1)
            wait_sc(ci, 0)
            issue_in(ci + 2, 0)
            wait_sc(ci + 1, 1)

            @pl.when(ci + 3 < rows_w)
            def _():
                issue_in(ci + 3, 1)

            return carry

        if rows_w % 2:
            lax.fori_loop(0, (rows_w - 1) // 2,
                          lambda i, cr: step(2 * i, cr), 0)
            wait_in(0)
            issue_sc(rows_w - 1, 0)
            wait_sc(rows_w - 1, 0)
        else:
            lax.fori_loop(0, (rows_w - 2) // 2,
                          lambda i, cr: step(2 * i, cr), 0)
            wait_in(0)
            issue_sc(rows_w - 2, 0)
            wait_in(1)
            issue_sc(rows_w - 1, 1)
            wait_sc(rows_w - 2, 0)
            wait_sc(rows_w - 1, 1)
        plsc.subcore_barrier()

        def wback(t, carry):
            r0 = s * rps + t * zr
            pltpu.sync_copy(acc.at[pl.ds(r0, zr)], stage)
            pltpu.sync_copy(stage, out_h.at[c, pl.ds(r0, zr)])
            return carry

        lax.fori_loop(0, rps // zr, wback, 0)

    return k(upd, src3d)


def _enc_w(p, pad_to=None):
    w1t = p["l1"]["W"].T
    if pad_to is not None:
        w1t = jnp.pad(w1t, ((0, pad_to - w1t.shape[0]), (0, 0)))
    return (w1t, p["l1"]["b"][None], p["l2"]["W"].T, p["l2"]["b"][None],
            p["g"][None], p["beta"][None])


def kernel(x, edge_attr, del_ext_force, spc_x, spc_edge_attr, V, params,
           edge_index, spc_edge_index):
    n = x.shape[0]
    n_pad = ((n + 16 * 128 - 1) // (16 * 128)) * (16 * 128)
    k_dim = spc_x.shape[0]
    e = edge_index.shape[1]
    ec = e // _NCH
    srcc = [edge_index[0, c * ec:(c + 1) * ec] for c in range(_NCH)]
    dstc = [edge_index[1, c * ec:(c + 1) * ec] for c in range(_NCH)]
    src3dc = [s.reshape(_NW, -1, _CE) for s in srcc]

    f_pad = jnp.pad(del_ext_force, ((0, 0), (0, 128 - del_ext_force.shape[1])))
    spc_x_pad = jnp.pad(spc_x, ((0, 0), (0, 128 - spc_x.shape[1])))
    sea_raw = jnp.pad(spc_edge_attr, ((0, 0), (0, 128 - spc_edge_attr.shape[1])))

    kk = jnp.arange(k_dim, dtype=jnp.int32)
    mdst = (spc_edge_index[1][:, None] == kk[None, :]).astype(F32)
    msrc = (spc_edge_index[0][:, None] == kk[None, :]).astype(F32)
    msrct = (kk[:, None] == spc_edge_index[0][None, :]).astype(F32)

    xe = _enc_apply(x, *_enc_w(params["node_encoder"]), _NB)
    se = _enc_apply(spc_x_pad, *_enc_w(params["spc_node_encoder"], 128), 128)

    q = _vt_x(V, xe)
    p_tab = _v_s(V, se)
    sx1 = jnp.concatenate([se, q], axis=1)

    sp = params["spc_proc"][0]
    sw1 = sp["edge_mlp"]["l1"]["W"]
    snw1 = sp["node_mlp"]["l1"]["W"]
    sxi1 = _spc_layer1(
        sx1, sea_raw, mdst, msrc, msrct,
        _enc_w(params["spc_edge_encoder"], 128),
        (sw1[:, :256].T, sw1[:, 256:512].T, sw1[:, 512:].T,
         sp["edge_mlp"]["l1"]["b"][None],
         sp["edge_mlp"]["l2"]["W"].T, sp["edge_mlp"]["l2"]["b"][None],
         sp["edge_mlp"]["g"][None], sp["edge_mlp"]["beta"][None],
         snw1[:, :256].T, snw1[:, 256:].T, sp["node_mlp"]["l1"]["b"][None],
         sp["node_mlp"]["l2"]["W"].T, sp["node_mlp"]["l2"]["b"][None],
         sp["node_mlp"]["g"][None], sp["node_mlp"]["beta"][None]))
    r_tab = _v_s(V, sxi1)

    p0 = params["proc"][0]
    w1 = p0["edge_mlp"]["l1"]["W"]
    a1, b1 = _ab_tables(xe, p_tab, w1[:, :128].T, w1[:, 128:256].T,
                        p0["edge_mlp"]["l1"]["b"][None],
                        w1[:, 256:384].T, w1[:, 384:512].T)
    upd1c = []
    part1s = []
    for c in range(_NCH):
        g1 = _sc_gather(a1, b1, dstc[c], srcc[c])
        u = _edge_mlp(edge_attr[c * ec:(c + 1) * ec], g1,
                      _enc_w(params["edge_encoder"]),
                      w1[:, 512:].T, p0["edge_mlp"]["l2"]["W"].T,
                      p0["edge_mlp"]["l2"]["b"][None],
                      p0["edge_mlp"]["g"][None],
                      p0["edge_mlp"]["beta"][None])
        upd1c.append(u)
        part1s.append(_sc_scatter(u, src3dc[c], n_pad))
    part1 = jnp.concatenate(part1s, axis=0)

    p1 = params["proc"][1]
    w1b = p1["edge_mlp"]["l1"]["W"]
    nw0 = p0["node_mlp"]["l1"]["W"]
    nodew0 = (nw0[:, :128].T, nw0[:, 128:256].T, nw0[:, 256:384].T,
              jnp.pad(nw0[:, 384:].T, ((0, 125), (0, 0))),
              p0["node_mlp"]["l1"]["b"][None],
              p0["node_mlp"]["l2"]["W"].T, p0["node_mlp"]["l2"]["b"][None],
              p0["node_mlp"]["g"][None], p0["node_mlp"]["beta"][None])
    xi1, a2, b2 = _node_mlp(
        xe, p_tab, part1, f_pad, nodew0,
        (w1b[:, :128].T, w1b[:, 128:256].T, p1["edge_mlp"]["l1"]["b"][None],
         w1b[:, 256:384].T, w1b[:, 384:512].T), "ab")

    part2s = []
    for c in range(_NCH):
        g2 = _sc_gather(a2, b2, dstc[c], srcc[c])
        u = _edge_mlp(upd1c[c], g2, None,
                      w1b[:, 512:].T, p1["edge_mlp"]["l2"]["W"].T,
                      p1["edge_mlp"]["l2"]["b"][None],
                      p1["edge_mlp"]["g"][None],
                      p1["edge_mlp"]["beta"][None])
        part2s.append(_sc_scatter(u, src3dc[c], n_pad))
    part2 = jnp.concatenate(part2s, axis=0)

    nw1 = p1["node_mlp"]["l1"]["W"]
    nodew1 = (nw1[:, :128].T, nw1[:, 128:256].T, nw1[:, 256:384].T,
              jnp.pad(nw1[:, 384:].T, ((0, 125), (0, 0))),
              p1["node_mlp"]["l1"]["b"][None],
              p1["node_mlp"]["l2"]["W"].T, p1["node_mlp"]["l2"]["b"][None],
              p1["node_mlp"]["g"][None], p1["node_mlp"]["beta"][None])
    wd1 = params["dec_l1"]["W"]
    wd2t = jnp.pad(params["dec_l2"]["W"].T, ((0, 0), (0, 125)))
    bd2 = jnp.pad(params["dec_l2"]["b"], (0, 125))[None]
    out_pad = _node_mlp(
        xi1, p_tab, part2, f_pad, nodew1,
        (r_tab, wd1[:, :128].T, wd1[:, 128:].T, params["dec_l1"]["b"][None],
         wd2t, bd2), "dec")
    return out_pad[:, :3]
